# Initial kernel scaffold; baseline (speedup 1.0000x reference)
import jax
import jax.numpy as jnp
from jax.experimental import pallas as pl


def kernel(x, edge_index, batch, y, degree, closeness, Wp1, Wp2, Wsm, bsm, Wc1, bc1, g1, be1, Wc2, bc2, g2, be2, Wc3, bc3, g3, be3, Wf1, bf1, Wf2, bf2):
    raise NotImplementedError("write your pallas kernel here")


# speedup vs baseline: 7.1964x; 7.1964x over previous
import functools

import jax
import jax.numpy as jnp
from jax import lax
from jax.experimental import pallas as pl
from jax.experimental.pallas import tpu as pltpu
from jax.experimental.pallas import tpu_sc as plsc

N = 10000
B = 20
NPG = 500
E = 160000
F = 256
D = 512
C = 10
M1 = 3
EPS = 1e-6

f32 = jnp.float32
i32 = jnp.int32
HI = lax.Precision.HIGHEST

NCORE = 2
NSUB = 16
EB = 125
EPT = E // NSUB
NBLK = EPT // EB
EPW = E // (NCORE * NSUB)
DBLK = EPW // EB
RB = 1000
NR = N // RB
RPT = N // NSUB


def _dot(a, b):
    return jnp.dot(a, b, precision=HI, preferred_element_type=f32)


def _score_body(x_ref, clo_ref, deg_ref, wp1_ref, wp2_ref, o_ref):
    xb = x_ref[0]
    t = jnp.tanh(_dot(xb, wp1_ref[...]))
    s = jnp.sum(t * wp2_ref[...], axis=1, keepdims=True)
    o_ref[0] = s + clo_ref[0] + deg_ref[0]


def _score_call(x3, clo3, deg3, wp1, wp2r):
    return pl.pallas_call(
        _score_body,
        grid=(B,),
        in_specs=[
            pl.BlockSpec((1, NPG, F), lambda b: (b, 0, 0)),
            pl.BlockSpec((1, NPG, 1), lambda b: (b, 0, 0)),
            pl.BlockSpec((1, NPG, 1), lambda b: (b, 0, 0)),
            pl.BlockSpec((F, F), lambda b: (0, 0)),
            pl.BlockSpec((1, F), lambda b: (0, 0)),
        ],
        out_specs=pl.BlockSpec((1, NPG, 1), lambda b: (b, 0, 0)),
        out_shape=jax.ShapeDtypeStruct((B, NPG, 1), f32),
    )(x3, clo3, deg3, wp1, wp2r)


def _top4_body(scp_ref, x_ref, kn_ref, idx_ref):
    sc = scp_ref[...]
    ii = lax.broadcasted_iota(i32, sc.shape, 1)
    iN = lax.broadcasted_iota(i32, (B, N), 1)
    boff = lax.broadcasted_iota(i32, (B, 1), 0) * NPG
    xall = x_ref[...]
    rows = []
    for j in range(M1 + 1):
        m = jnp.max(sc, axis=1, keepdims=True)
        cand = jnp.where(sc == m, ii, jnp.int32(2**30))
        idx = jnp.min(cand, axis=1, keepdims=True)
        gate = jnp.tanh(m)
        oh = jnp.where(iN == idx + boff, gate, 0.0)
        rows.append(_dot(oh, xall))
        sc = jnp.where(ii == idx, jnp.float32(-3e38), sc)
        idx_ref[:, j, :] = idx
    r0, r1, r2, r3 = rows
    a, b_ = jnp.minimum(r0, r1), jnp.maximum(r0, r1)
    c_, d_ = jnp.minimum(r2, r3), jnp.maximum(r2, r3)
    a, c_ = jnp.minimum(a, c_), jnp.maximum(a, c_)
    b_, d_ = jnp.minimum(b_, d_), jnp.maximum(b_, d_)
    b_, c_ = jnp.minimum(b_, c_), jnp.maximum(b_, c_)
    kn_ref[:, 0, :] = a
    kn_ref[:, 1, :] = b_
    kn_ref[:, 2, :] = c_
    kn_ref[:, 3, :] = d_


def _top4_call(scp, x):
    return pl.pallas_call(
        _top4_body,
        out_shape=(
            jax.ShapeDtypeStruct((B, M1 + 1, F), f32),
            jax.ShapeDtypeStruct((B, M1 + 1, 1), i32),
        ),
    )(scp, x)


def _spline_body(x_ref, kn_ref, idx_ref, deg_ref, crow_ref,
                 wx_ref, w0_ref, w1_ref, w2_ref, h1p_ref, x1_ref):
    xg = x_ref[0]
    kn = kn_ref[0]
    last = kn[3:4, :]
    pl3 = jnp.maximum(xg - last, 0.0) ** 3
    h = _dot(xg, wx_ref[...])
    for k, wr in enumerate((w0_ref, w1_ref, w2_ref)):
        kk = kn[k:k + 1, :]
        dk = (jnp.maximum(xg - kk, 0.0) ** 3 - pl3) / (last - kk + EPS)
        h = h + _dot(dk, wr[...])
    h = h + crow_ref[...]
    idxb = idx_ref[0]
    oh = (lax.broadcasted_iota(i32, (M1 + 1, NPG), 1) == idxb).astype(f32)
    hr = _dot(oh, h)
    x1_ref[0] = jnp.max(hr, axis=0, keepdims=True)
    d = deg_ref[0]
    dv = lax.rsqrt(1.0 + d[:, 0:1] + d[:, 1:2])
    hp = h * dv
    h1p_ref[0, 0] = hp[:, 0:128]
    h1p_ref[1, 0] = hp[:, 128:256]


def _spline_call(x3, kn, idx4, degpB, crow, wx, w0, w1, w2):
    return pl.pallas_call(
        _spline_body,
        grid=(B,),
        in_specs=[
            pl.BlockSpec((1, NPG, F), lambda b: (b, 0, 0)),
            pl.BlockSpec((1, M1 + 1, F), lambda b: (b, 0, 0)),
            pl.BlockSpec((1, M1 + 1, 1), lambda b: (b, 0, 0)),
            pl.BlockSpec((1, NPG, 2), lambda b: (b, 0, 0)),
            pl.BlockSpec((1, F), lambda b: (0, 0)),
            pl.BlockSpec((F, F), lambda b: (0, 0)),
            pl.BlockSpec((F, F), lambda b: (0, 0)),
            pl.BlockSpec((F, F), lambda b: (0, 0)),
            pl.BlockSpec((F, F), lambda b: (0, 0)),
        ],
        out_specs=(
            pl.BlockSpec((2, 1, NPG, 128), lambda b: (0, b, 0, 0)),
            pl.BlockSpec((1, 1, F), lambda b: (b, 0, 0)),
        ),
        out_shape=(
            jax.ShapeDtypeStruct((2, B, NPG, 128), f32),
            jax.ShapeDtypeStruct((B, 1, F), f32),
        ),
    )(x3, kn, idx4, degpB, crow, wx, w0, w1, w2)


def _make_deg_kernel():
    mesh = plsc.VectorSubcoreMesh(core_axis_name="c", subcore_axis_name="s")

    @functools.partial(
        pl.kernel,
        out_type=jax.ShapeDtypeStruct((NCORE, N), f32),
        mesh=mesh,
        scratch_types=[
            pltpu.VMEM((DBLK, EB), i32),
            pltpu.VMEM((EB,), f32),
            pltpu.VMEM_SHARED((N,), f32),
            pltpu.SemaphoreType.DMA,
        ],
    )
    def deg_kernel(dst_hbm, ones_hbm, zeros_hbm, out_hbm, dst_v, ones_v,
                   acc_sh, sem):
        c = lax.axis_index("c")
        s = lax.axis_index("s")
        wid = c * NSUB + s
        pltpu.sync_copy(dst_hbm.at[wid], dst_v)
        pltpu.sync_copy(ones_hbm, ones_v)

        @pl.when(s == 0)
        def _():
            pltpu.sync_copy(zeros_hbm, acc_sh)

        plsc.subcore_barrier()

        def body(j, carry):
            pltpu.sync_copy(ones_v, acc_sh.at[dst_v.at[j]], add=True)
            return carry

        lax.fori_loop(0, DBLK, body, 0)
        plsc.subcore_barrier()

        @pl.when(s == 0)
        def _():
            pltpu.sync_copy(acc_sh, out_hbm.at[c])

    return deg_kernel


CH0 = 632
CH1 = N - CH0 * (NSUB - 1)


def _make_scatter_kernel(nch):
    cpc = nch // NCORE
    mesh = plsc.VectorSubcoreMesh(core_axis_name="c", subcore_axis_name="s")

    @functools.partial(
        pl.kernel,
        out_type=jax.ShapeDtypeStruct((nch * N, 128), f32),
        mesh=mesh,
        scratch_types=[
            pltpu.VMEM((NBLK, EB), i32),
            pltpu.VMEM((NBLK, EB), i32),
            pltpu.VMEM((EB, 128), f32),
            pltpu.VMEM_SHARED((N, 128), f32),
            pltpu.SemaphoreType.DMA,
        ],
    )
    def scat_kernel(hp_hbm, srcg_hbm, dst_hbm, out_hbm, src_v, dst_v, buf,
                    acc_sh, sem):
        c = lax.axis_index("c")
        s = lax.axis_index("s")
        pltpu.sync_copy(dst_hbm.at[s], dst_v)
        for t in range(cpc):
            q = c * cpc + t
            pltpu.sync_copy(srcg_hbm.at[q, s], src_v)

            @pl.when(s < NSUB - 1)
            def _():
                st = pl.multiple_of(q * N + s * CH0, 8)
                sl = pl.multiple_of(s * CH0, 8)
                pltpu.sync_copy(hp_hbm.at[pl.ds(st, CH0)],
                                acc_sh.at[pl.ds(sl, CH0)])

            @pl.when(s == NSUB - 1)
            def _():
                st = pl.multiple_of(q * N + CH0 * (NSUB - 1), 8)
                pltpu.sync_copy(hp_hbm.at[pl.ds(st, CH1)],
                                acc_sh.at[pl.ds(CH0 * (NSUB - 1), CH1)])

            plsc.subcore_barrier()

            def body(j, carry):
                pltpu.async_copy(hp_hbm.at[src_v.at[j]], buf, sem).wait()
                pltpu.sync_copy(buf, acc_sh.at[dst_v.at[j]], add=True)
                return carry

            lax.fori_loop(0, NBLK, body, 0)
            plsc.subcore_barrier()

            @pl.when(s < NSUB - 1)
            def _():
                st = pl.multiple_of(q * N + s * CH0, 8)
                sl = pl.multiple_of(s * CH0, 8)
                pltpu.sync_copy(acc_sh.at[pl.ds(sl, CH0)],
                                out_hbm.at[pl.ds(st, CH0)])

            @pl.when(s == NSUB - 1)
            def _():
                st = pl.multiple_of(q * N + CH0 * (NSUB - 1), 8)
                pltpu.sync_copy(acc_sh.at[pl.ds(CH0 * (NSUB - 1), CH1)],
                                out_hbm.at[pl.ds(st, CH1)])

            plsc.subcore_barrier()

    return scat_kernel


def _post1_body(agg_ref, deg_ref, w_ref, bc_ref, y_ref, st_ref, tacc, sacc):
    r = pl.program_id(1)
    ci = pl.program_id(2)
    d = deg_ref[0]
    dv = lax.rsqrt(1.0 + d[:, 0:1] + d[:, 1:2])
    part = _dot(agg_ref[0, 0] * dv, w_ref[0, 0])

    @pl.when(ci == 0)
    def _():
        tacc[...] = part

    @pl.when(ci > 0)
    def _():
        tacc[...] = tacc[...] + part

    @pl.when(ci == 1)
    def _():
        yv = jnp.maximum(tacc[...] + bc_ref[0], 0.0)
        y_ref[0, 0] = yv
        s1 = jnp.sum(yv, axis=0, keepdims=True)
        s2 = jnp.sum(yv * yv, axis=0, keepdims=True)

        @pl.when(r == 0)
        def _():
            sacc[0:1] = s1
            sacc[1:2] = s2

        @pl.when(r > 0)
        def _():
            sacc[0:1] = sacc[0:1] + s1
            sacc[1:2] = sacc[1:2] + s2

        @pl.when(r == NR - 1)
        def _():
            st_ref[0] = sacc[0:2]


def _post1_call(agg1, degpR, wc1r, bc1r):
    return pl.pallas_call(
        _post1_body,
        grid=(4, NR, 2),
        in_specs=[
            pl.BlockSpec((1, 1, RB, 128), lambda co, r, ci: (ci, r, 0, 0)),
            pl.BlockSpec((1, RB, 2), lambda co, r, ci: (r, 0, 0)),
            pl.BlockSpec((1, 1, 128, 128), lambda co, r, ci: (ci, co, 0, 0)),
            pl.BlockSpec((1, 1, 128), lambda co, r, ci: (co, 0, 0)),
        ],
        out_specs=(
            pl.BlockSpec((1, 1, RB, 128), lambda co, r, ci: (co, r, 0, 0)),
            pl.BlockSpec((1, 2, 128), lambda co, r, ci: (co, 0, 0)),
        ),
        out_shape=(
            jax.ShapeDtypeStruct((4, NR, RB, 128), f32),
            jax.ShapeDtypeStruct((4, 2, 128), f32),
        ),
        scratch_shapes=[
            pltpu.VMEM((RB, 128), f32),
            pltpu.VMEM((8, 128), f32),
        ],
    )(agg1, degpR, wc1r, bc1r)


def _mm_body(y_ref, st_ref, g_ref, be_ref, deg_ref, w_ref, o_ref, tacc):
    ci = pl.program_id(2)
    st = st_ref[0]
    m = st[0:1] * (1.0 / N)
    v = st[1:2] * (1.0 / N) - m * m
    a = g_ref[0] * lax.rsqrt(v + 1e-5)
    cs = be_ref[0] - m * a
    xbn = y_ref[0, 0] * a + cs
    part = _dot(xbn, w_ref[0, 0])

    @pl.when(ci == 0)
    def _():
        tacc[...] = part

    @pl.when(ci > 0)
    def _():
        tacc[...] = tacc[...] + part

    @pl.when(ci == 3)
    def _():
        d = deg_ref[0]
        dv = lax.rsqrt(1.0 + d[:, 0:1] + d[:, 1:2])
        o_ref[0, 0] = tacc[...] * dv


def _mm_call(yl, st, gr, ber, degpR, wr):
    return pl.pallas_call(
        _mm_body,
        grid=(4, NR, 4),
        in_specs=[
            pl.BlockSpec((1, 1, RB, 128), lambda co, r, ci: (ci, r, 0, 0)),
            pl.BlockSpec((1, 2, 128), lambda co, r, ci: (ci, 0, 0)),
            pl.BlockSpec((1, 1, 128), lambda co, r, ci: (ci, 0, 0)),
            pl.BlockSpec((1, 1, 128), lambda co, r, ci: (ci, 0, 0)),
            pl.BlockSpec((1, RB, 2), lambda co, r, ci: (r, 0, 0)),
            pl.BlockSpec((1, 1, 128, 128), lambda co, r, ci: (ci, co, 0, 0)),
        ],
        out_specs=pl.BlockSpec((1, 1, RB, 128), lambda co, r, ci: (co, r, 0, 0)),
        out_shape=jax.ShapeDtypeStruct((4, NR, RB, 128), f32),
        scratch_shapes=[pltpu.VMEM((RB, 128), f32)],
    )(yl, st, gr, ber, degpR, wr)


def _post23_body(agg_ref, bc_ref, deg_ref, y_ref, st_ref, sacc):
    r = pl.program_id(1)
    d = deg_ref[0]
    dv = lax.rsqrt(1.0 + d[:, 0:1] + d[:, 1:2])
    yv = jnp.maximum(agg_ref[0, 0] * dv + bc_ref[0], 0.0)
    y_ref[0, 0] = yv
    s1 = jnp.sum(yv, axis=0, keepdims=True)
    s2 = jnp.sum(yv * yv, axis=0, keepdims=True)

    @pl.when(r == 0)
    def _():
        sacc[0:1] = s1
        sacc[1:2] = s2

    @pl.when(r > 0)
    def _():
        sacc[0:1] = sacc[0:1] + s1
        sacc[1:2] = sacc[1:2] + s2

    @pl.when(r == NR - 1)
    def _():
        st_ref[0] = sacc[0:2]


def _post23_call(agg, bcr, degpR):
    return pl.pallas_call(
        _post23_body,
        grid=(4, NR),
        in_specs=[
            pl.BlockSpec((1, 1, RB, 128), lambda co, r: (co, r, 0, 0)),
            pl.BlockSpec((1, 1, 128), lambda co, r: (co, 0, 0)),
            pl.BlockSpec((1, RB, 2), lambda co, r: (r, 0, 0)),
        ],
        out_specs=(
            pl.BlockSpec((1, 1, RB, 128), lambda co, r: (co, r, 0, 0)),
            pl.BlockSpec((1, 2, 128), lambda co, r: (co, 0, 0)),
        ),
        out_shape=(
            jax.ShapeDtypeStruct((4, NR, RB, 128), f32),
            jax.ShapeDtypeStruct((4, 2, 128), f32),
        ),
        scratch_shapes=[pltpu.VMEM((8, 128), f32)],
    )(agg, bcr, degpR)


def _final_body(y_ref, st_ref, g_ref, be_ref, x1_ref, wf1_ref, bf1_ref,
                wf2_ref, bf2_ref, o_ref):
    parts = []
    for cc in range(4):
        st = st_ref[cc]
        m = st[0:1] * (1.0 / N)
        v = st[1:2] * (1.0 / N) - m * m
        a = g_ref[cc] * lax.rsqrt(v + 1e-5)
        cs = be_ref[cc] - m * a
        yb = y_ref[cc, 0]
        hg_c = a * (jnp.sum(yb, axis=0, keepdims=True) * (1.0 / NPG)) + cs
        parts.append(hg_c)
    hg = jnp.concatenate(parts, axis=1)
    z = jnp.concatenate([hg, x1_ref[0]], axis=1)
    z = jnp.maximum(_dot(z, wf1_ref[...]) + bf1_ref[...], 0.0)
    z = _dot(z, wf2_ref[...]) + bf2_ref[...]
    mz = jnp.max(z, axis=1, keepdims=True)
    lse = jnp.log(jnp.sum(jnp.exp(z - mz), axis=1, keepdims=True)) + mz
    o_ref[0] = z - lse


def _final_call(y3g, st3, g3r, be3r, x1, wf1, bf1r, wf2, bf2r):
    return pl.pallas_call(
        _final_body,
        grid=(B,),
        in_specs=[
            pl.BlockSpec((4, 1, NPG, 128), lambda b: (0, b, 0, 0)),
            pl.BlockSpec((4, 2, 128), lambda b: (0, 0, 0)),
            pl.BlockSpec((4, 1, 128), lambda b: (0, 0, 0)),
            pl.BlockSpec((4, 1, 128), lambda b: (0, 0, 0)),
            pl.BlockSpec((1, 1, F), lambda b: (b, 0, 0)),
            pl.BlockSpec((D + F, D), lambda b: (0, 0)),
            pl.BlockSpec((1, D), lambda b: (0, 0)),
            pl.BlockSpec((D, C), lambda b: (0, 0)),
            pl.BlockSpec((1, C), lambda b: (0, 0)),
        ],
        out_specs=pl.BlockSpec((1, 1, C), lambda b: (b, 0, 0)),
        out_shape=jax.ShapeDtypeStruct((B, 1, C), f32),
    )(y3g, st3, g3r, be3r, x1, wf1, bf1r, wf2, bf2r)


@functools.cache
def _get_deg_kernel():
    return _make_deg_kernel()


@functools.cache
def _get_scatter_kernel(nch):
    return _make_scatter_kernel(nch)


def kernel(x, edge_index, batch, y, degree, closeness, Wp1, Wp2, Wsm, bsm,
           Wc1, bc1, g1, be1, Wc2, bc2, g2, be2, Wc3, bc3, g3, be3,
           Wf1, bf1, Wf2, bf2):
    x3 = x.reshape(B, NPG, F)
    clo3 = closeness.reshape(B, NPG, 1)
    degin3 = degree.reshape(B, NPG, 1)
    wp2r = Wp2.reshape(1, F)

    score = _score_call(x3, clo3, degin3, Wp1, wp2r)
    scp = jnp.concatenate(
        [score.reshape(B, NPG), jnp.full((B, 12), -3e38, f32)], axis=1)
    kn, idx4 = _top4_call(scp, x)

    src = edge_index[0]
    dst = edge_index[1]
    dst32 = dst.reshape(NCORE * NSUB, DBLK, EB)
    ones_eb = jnp.ones((EB,), f32)
    zeros_n = jnp.zeros((N,), f32)
    degp = _get_deg_kernel()(dst32, ones_eb, zeros_n)
    degp2 = jnp.moveaxis(degp, 0, 1)
    degpB = degp2.reshape(B, NPG, 2)
    degpR = degp2.reshape(NR, RB, 2)

    crow = Wsm[0:1] + bsm[None, :]
    wx = Wsm[1:1 + F]
    w0 = Wsm[1 + F:1 + 2 * F]
    w1 = Wsm[1 + 2 * F:1 + 3 * F]
    w2 = Wsm[1 + 3 * F:1 + 4 * F]
    h1p, x1 = _spline_call(x3, kn, idx4, degpB, crow, wx, w0, w1, w2)

    dst16 = dst.reshape(NSUB, NBLK, EB)
    srcg2 = (src[None, :].astype(i32)
             + (jnp.arange(2, dtype=i32) * N)[:, None]).reshape(
                 2, NSUB, NBLK, EB)
    srcg4 = (src[None, :].astype(i32)
             + (jnp.arange(4, dtype=i32) * N)[:, None]).reshape(
                 4, NSUB, NBLK, EB)

    agg1 = _get_scatter_kernel(2)(h1p.reshape(2 * N, 128), srcg2, dst16)
    agg1 = agg1.reshape(2, NR, RB, 128)

    wc1r = Wc1.reshape(2, 128, 4, 128).transpose(0, 2, 1, 3)
    bc1r = bc1.reshape(4, 1, 128)
    y1, st1 = _post1_call(agg1, degpR, wc1r, bc1r)

    wc2r = Wc2.reshape(4, 128, 4, 128).transpose(0, 2, 1, 3)
    h2p2 = _mm_call(y1, st1, g1.reshape(4, 1, 128), be1.reshape(4, 1, 128),
                    degpR, wc2r)
    agg2 = _get_scatter_kernel(4)(h2p2.reshape(4 * N, 128), srcg4, dst16)
    y2, st2 = _post23_call(agg2.reshape(4, NR, RB, 128),
                           bc2.reshape(4, 1, 128), degpR)

    wc3r = Wc3.reshape(4, 128, 4, 128).transpose(0, 2, 1, 3)
    h2p3 = _mm_call(y2, st2, g2.reshape(4, 1, 128), be2.reshape(4, 1, 128),
                    degpR, wc3r)
    agg3 = _get_scatter_kernel(4)(h2p3.reshape(4 * N, 128), srcg4, dst16)
    y3, st3 = _post23_call(agg3.reshape(4, NR, RB, 128),
                           bc3.reshape(4, 1, 128), degpR)

    out = _final_call(y3.reshape(4, B, NPG, 128), st3,
                      g3.reshape(4, 1, 128), be3.reshape(4, 1, 128),
                      x1, Wf1, bf1.reshape(1, D), Wf2, bf2.reshape(1, C))
    return out.reshape(B, C)


# speedup vs baseline: 10.1225x; 1.4066x over previous
import functools

import jax
import jax.numpy as jnp
from jax import lax
from jax.experimental import pallas as pl
from jax.experimental.pallas import tpu as pltpu
from jax.experimental.pallas import tpu_sc as plsc

N = 10000
B = 20
NPG = 500
E = 160000
F = 256
D = 512
C = 10
M1 = 3
EPS = 1e-6

f32 = jnp.float32
i32 = jnp.int32
HI = lax.Precision.HIGHEST

NCORE = 2
NSUB = 16
EB = 100
EPT = E // NSUB
NBLK = EPT // EB
NBLK2 = NBLK // 2
EPW = E // (NCORE * NSUB)
DBLK = EPW // EB
RB = 1000
NR = N // RB
RPT = N // NSUB


def _dot(a, b):
    return jnp.dot(a, b, precision=HI, preferred_element_type=f32)


def _doth(a, b):
    return jnp.dot(a, b, precision=lax.Precision.DEFAULT,
                   preferred_element_type=f32)


def _score_body(x_ref, clo_ref, deg_ref, wp1_ref, wp2_ref, o_ref):
    xb = x_ref[0]
    t = jnp.tanh(_dot(xb, wp1_ref[...]))
    s = jnp.sum(t * wp2_ref[...], axis=1, keepdims=True)
    o_ref[0] = s + clo_ref[0] + deg_ref[0]


def _score_call(x3, clo3, deg3, wp1, wp2r):
    return pl.pallas_call(
        _score_body,
        grid=(B,),
        in_specs=[
            pl.BlockSpec((1, NPG, F), lambda b: (b, 0, 0)),
            pl.BlockSpec((1, NPG, 1), lambda b: (b, 0, 0)),
            pl.BlockSpec((1, NPG, 1), lambda b: (b, 0, 0)),
            pl.BlockSpec((F, F), lambda b: (0, 0)),
            pl.BlockSpec((1, F), lambda b: (0, 0)),
        ],
        out_specs=pl.BlockSpec((1, NPG, 1), lambda b: (b, 0, 0)),
        out_shape=jax.ShapeDtypeStruct((B, NPG, 1), f32),
    )(x3, clo3, deg3, wp1, wp2r)


def _top4_body(scp_ref, x_ref, kn_ref, idx_ref):
    sc = scp_ref[...]
    ii = lax.broadcasted_iota(i32, sc.shape, 1)
    iN = lax.broadcasted_iota(i32, (B, N), 1)
    boff = lax.broadcasted_iota(i32, (B, 1), 0) * NPG
    xall = x_ref[...]
    rows = []
    for j in range(M1 + 1):
        m = jnp.max(sc, axis=1, keepdims=True)
        cand = jnp.where(sc == m, ii, jnp.int32(2**30))
        idx = jnp.min(cand, axis=1, keepdims=True)
        gate = jnp.tanh(m)
        oh = jnp.where(iN == idx + boff, gate, 0.0)
        rows.append(_dot(oh, xall))
        sc = jnp.where(ii == idx, jnp.float32(-3e38), sc)
        idx_ref[:, j, :] = idx
    r0, r1, r2, r3 = rows
    a, b_ = jnp.minimum(r0, r1), jnp.maximum(r0, r1)
    c_, d_ = jnp.minimum(r2, r3), jnp.maximum(r2, r3)
    a, c_ = jnp.minimum(a, c_), jnp.maximum(a, c_)
    b_, d_ = jnp.minimum(b_, d_), jnp.maximum(b_, d_)
    b_, c_ = jnp.minimum(b_, c_), jnp.maximum(b_, c_)
    kn_ref[:, 0, :] = a
    kn_ref[:, 1, :] = b_
    kn_ref[:, 2, :] = c_
    kn_ref[:, 3, :] = d_


def _top4_call(scp, x):
    return pl.pallas_call(
        _top4_body,
        out_shape=(
            jax.ShapeDtypeStruct((B, M1 + 1, F), f32),
            jax.ShapeDtypeStruct((B, M1 + 1, 1), i32),
        ),
    )(scp, x)


def _spline_body(x_ref, kn_ref, idx_ref, deg_ref, crow_ref,
                 wx_ref, w0_ref, w1_ref, w2_ref, h1p_ref, x1_ref):
    xg = x_ref[0]
    kn = kn_ref[0]
    last = kn[3:4, :]
    pl3 = jnp.maximum(xg - last, 0.0) ** 3
    h = _doth(xg, wx_ref[...])
    for k, wr in enumerate((w0_ref, w1_ref, w2_ref)):
        kk = kn[k:k + 1, :]
        dk = (jnp.maximum(xg - kk, 0.0) ** 3 - pl3) / (last - kk + EPS)
        h = h + _doth(dk, wr[...])
    h = h + crow_ref[...]
    idxb = idx_ref[0]
    oh = (lax.broadcasted_iota(i32, (M1 + 1, NPG), 1) == idxb).astype(f32)
    hr = _dot(oh, h)
    x1_ref[0] = jnp.max(hr, axis=0, keepdims=True)
    d = deg_ref[0]
    dv = lax.rsqrt(1.0 + d[:, 0:1] + d[:, 1:2])
    hp = h * dv
    h1p_ref[0, 0] = hp[:, 0:128]
    h1p_ref[1, 0] = hp[:, 128:256]


def _spline_call(x3, kn, idx4, degpB, crow, wx, w0, w1, w2):
    return pl.pallas_call(
        _spline_body,
        grid=(B,),
        in_specs=[
            pl.BlockSpec((1, NPG, F), lambda b: (b, 0, 0)),
            pl.BlockSpec((1, M1 + 1, F), lambda b: (b, 0, 0)),
            pl.BlockSpec((1, M1 + 1, 1), lambda b: (b, 0, 0)),
            pl.BlockSpec((1, NPG, 2), lambda b: (b, 0, 0)),
            pl.BlockSpec((1, F), lambda b: (0, 0)),
            pl.BlockSpec((F, F), lambda b: (0, 0)),
            pl.BlockSpec((F, F), lambda b: (0, 0)),
            pl.BlockSpec((F, F), lambda b: (0, 0)),
            pl.BlockSpec((F, F), lambda b: (0, 0)),
        ],
        out_specs=(
            pl.BlockSpec((2, 1, NPG, 128), lambda b: (0, b, 0, 0)),
            pl.BlockSpec((1, 1, F), lambda b: (b, 0, 0)),
        ),
        out_shape=(
            jax.ShapeDtypeStruct((2, B, NPG, 128), f32),
            jax.ShapeDtypeStruct((B, 1, F), f32),
        ),
    )(x3, kn, idx4, degpB, crow, wx, w0, w1, w2)


def _make_deg_kernel():
    mesh = plsc.VectorSubcoreMesh(core_axis_name="c", subcore_axis_name="s")

    @functools.partial(
        pl.kernel,
        out_type=jax.ShapeDtypeStruct((NCORE, N), f32),
        mesh=mesh,
        scratch_types=[
            pltpu.VMEM((DBLK, EB), i32),
            pltpu.VMEM((EB,), f32),
            pltpu.VMEM_SHARED((N,), f32),
            pltpu.SemaphoreType.DMA,
        ],
    )
    def deg_kernel(dst_hbm, ones_hbm, zeros_hbm, out_hbm, dst_v, ones_v,
                   acc_sh, sem):
        c = lax.axis_index("c")
        s = lax.axis_index("s")
        wid = c * NSUB + s
        pltpu.sync_copy(dst_hbm.at[wid], dst_v)
        pltpu.sync_copy(ones_hbm, ones_v)

        @pl.when(s == 0)
        def _():
            pltpu.sync_copy(zeros_hbm, acc_sh)

        plsc.subcore_barrier()

        def body(j, carry):
            pltpu.sync_copy(ones_v, acc_sh.at[dst_v.at[j]], add=True)
            return carry

        lax.fori_loop(0, DBLK, body, 0)
        plsc.subcore_barrier()

        @pl.when(s == 0)
        def _():
            pltpu.sync_copy(acc_sh, out_hbm.at[c])

    return deg_kernel


CH0 = 632
CH1 = N - CH0 * (NSUB - 1)


def _make_scatter_kernel(nch):
    cpc = nch // NCORE
    mesh = plsc.VectorSubcoreMesh(core_axis_name="c", subcore_axis_name="s")

    @functools.partial(
        pl.kernel,
        out_type=jax.ShapeDtypeStruct((nch * N, 128), f32),
        mesh=mesh,
        scratch_types=[
            pltpu.VMEM((NBLK2, EB), i32),
            pltpu.VMEM((NBLK2, EB), i32),
            pltpu.VMEM((EB, 128), f32),
            pltpu.VMEM((EB, 128), f32),
            pltpu.VMEM_SHARED((N, 128), f32),
            pltpu.SemaphoreType.DMA,
            pltpu.SemaphoreType.DMA,
        ],
    )
    def scat_kernel(hp_hbm, srcg_hbm, dst_hbm, out_hbm, src_v, dst_v, buf0,
                    buf1, acc_sh, sem0, sem1):
        c = lax.axis_index("c")
        s = lax.axis_index("s")
        for t in range(cpc):
            q = c * cpc + t
            @pl.when(s < NSUB - 1)
            def _():
                st = pl.multiple_of(q * N + s * CH0, 8)
                sl = pl.multiple_of(s * CH0, 8)
                pltpu.sync_copy(hp_hbm.at[pl.ds(st, CH0)],
                                acc_sh.at[pl.ds(sl, CH0)])

            @pl.when(s == NSUB - 1)
            def _():
                st = pl.multiple_of(q * N + CH0 * (NSUB - 1), 8)
                pltpu.sync_copy(hp_hbm.at[pl.ds(st, CH1)],
                                acc_sh.at[pl.ds(CH0 * (NSUB - 1), CH1)])

            plsc.subcore_barrier()

            for half in range(2):
                pltpu.sync_copy(srcg_hbm.at[q, s, half], src_v)
                pltpu.sync_copy(dst_hbm.at[s, half], dst_v)
                pltpu.async_copy(hp_hbm.at[src_v.at[0]], buf0, sem0)

                def body(i, carry):
                    j0 = i * 2
                    j1 = j0 + 1
                    pltpu.async_copy(hp_hbm.at[src_v.at[j1]], buf1, sem1)
                    pltpu.make_async_copy(hp_hbm.at[src_v.at[j0]], buf0,
                                          sem0).wait()
                    pltpu.sync_copy(buf0, acc_sh.at[dst_v.at[j0]], add=True)

                    @pl.when(j1 + 1 < NBLK2)
                    def _():
                        pltpu.async_copy(hp_hbm.at[src_v.at[j1 + 1]], buf0,
                                         sem0)

                    pltpu.make_async_copy(hp_hbm.at[src_v.at[j1]], buf1,
                                          sem1).wait()
                    pltpu.sync_copy(buf1, acc_sh.at[dst_v.at[j1]], add=True)
                    return carry

                lax.fori_loop(0, NBLK2 // 2, body, 0)
            plsc.subcore_barrier()

            @pl.when(s < NSUB - 1)
            def _():
                st = pl.multiple_of(q * N + s * CH0, 8)
                sl = pl.multiple_of(s * CH0, 8)
                pltpu.sync_copy(acc_sh.at[pl.ds(sl, CH0)],
                                out_hbm.at[pl.ds(st, CH0)])

            @pl.when(s == NSUB - 1)
            def _():
                st = pl.multiple_of(q * N + CH0 * (NSUB - 1), 8)
                pltpu.sync_copy(acc_sh.at[pl.ds(CH0 * (NSUB - 1), CH1)],
                                out_hbm.at[pl.ds(st, CH1)])

            plsc.subcore_barrier()

    return scat_kernel


def _post1_body(agg_ref, deg_ref, w_ref, bc_ref, y_ref, st_ref, tacc, sacc):
    r = pl.program_id(1)
    ci = pl.program_id(2)
    d = deg_ref[0]
    dv = lax.rsqrt(1.0 + d[:, 0:1] + d[:, 1:2])
    part = _doth(agg_ref[0, 0] * dv, w_ref[0, 0])

    @pl.when(ci == 0)
    def _():
        tacc[...] = part

    @pl.when(ci > 0)
    def _():
        tacc[...] = tacc[...] + part

    @pl.when(ci == 1)
    def _():
        yv = jnp.maximum(tacc[...] + bc_ref[0], 0.0)
        y_ref[0, 0] = yv
        s1 = jnp.sum(yv, axis=0, keepdims=True)
        s2 = jnp.sum(yv * yv, axis=0, keepdims=True)

        @pl.when(r == 0)
        def _():
            sacc[0:1] = s1
            sacc[1:2] = s2

        @pl.when(r > 0)
        def _():
            sacc[0:1] = sacc[0:1] + s1
            sacc[1:2] = sacc[1:2] + s2

        @pl.when(r == NR - 1)
        def _():
            st_ref[0] = sacc[0:2]


def _post1_call(agg1, degpR, wc1r, bc1r):
    return pl.pallas_call(
        _post1_body,
        grid=(4, NR, 2),
        in_specs=[
            pl.BlockSpec((1, 1, RB, 128), lambda co, r, ci: (ci, r, 0, 0)),
            pl.BlockSpec((1, RB, 2), lambda co, r, ci: (r, 0, 0)),
            pl.BlockSpec((1, 1, 128, 128), lambda co, r, ci: (ci, co, 0, 0)),
            pl.BlockSpec((1, 1, 128), lambda co, r, ci: (co, 0, 0)),
        ],
        out_specs=(
            pl.BlockSpec((1, 1, RB, 128), lambda co, r, ci: (co, r, 0, 0)),
            pl.BlockSpec((1, 2, 128), lambda co, r, ci: (co, 0, 0)),
        ),
        out_shape=(
            jax.ShapeDtypeStruct((4, NR, RB, 128), f32),
            jax.ShapeDtypeStruct((4, 2, 128), f32),
        ),
        scratch_shapes=[
            pltpu.VMEM((RB, 128), f32),
            pltpu.VMEM((8, 128), f32),
        ],
    )(agg1, degpR, wc1r, bc1r)


def _mm_body(y_ref, st_ref, g_ref, be_ref, deg_ref, w_ref, o_ref, tacc):
    ci = pl.program_id(2)
    st = st_ref[0]
    m = st[0:1] * (1.0 / N)
    v = st[1:2] * (1.0 / N) - m * m
    a = g_ref[0] * lax.rsqrt(v + 1e-5)
    cs = be_ref[0] - m * a
    xbn = y_ref[0, 0] * a + cs
    part = _doth(xbn, w_ref[0, 0])

    @pl.when(ci == 0)
    def _():
        tacc[...] = part

    @pl.when(ci > 0)
    def _():
        tacc[...] = tacc[...] + part

    @pl.when(ci == 3)
    def _():
        d = deg_ref[0]
        dv = lax.rsqrt(1.0 + d[:, 0:1] + d[:, 1:2])
        o_ref[0, 0] = tacc[...] * dv


def _mm_call(yl, st, gr, ber, degpR, wr):
    return pl.pallas_call(
        _mm_body,
        grid=(4, NR, 4),
        in_specs=[
            pl.BlockSpec((1, 1, RB, 128), lambda co, r, ci: (ci, r, 0, 0)),
            pl.BlockSpec((1, 2, 128), lambda co, r, ci: (ci, 0, 0)),
            pl.BlockSpec((1, 1, 128), lambda co, r, ci: (ci, 0, 0)),
            pl.BlockSpec((1, 1, 128), lambda co, r, ci: (ci, 0, 0)),
            pl.BlockSpec((1, RB, 2), lambda co, r, ci: (r, 0, 0)),
            pl.BlockSpec((1, 1, 128, 128), lambda co, r, ci: (ci, co, 0, 0)),
        ],
        out_specs=pl.BlockSpec((1, 1, RB, 128), lambda co, r, ci: (co, r, 0, 0)),
        out_shape=jax.ShapeDtypeStruct((4, NR, RB, 128), f32),
        scratch_shapes=[pltpu.VMEM((RB, 128), f32)],
    )(yl, st, gr, ber, degpR, wr)


def _post23_body(agg_ref, bc_ref, deg_ref, y_ref, st_ref, sacc):
    r = pl.program_id(1)
    d = deg_ref[0]
    dv = lax.rsqrt(1.0 + d[:, 0:1] + d[:, 1:2])
    yv = jnp.maximum(agg_ref[0, 0] * dv + bc_ref[0], 0.0)
    y_ref[0, 0] = yv
    s1 = jnp.sum(yv, axis=0, keepdims=True)
    s2 = jnp.sum(yv * yv, axis=0, keepdims=True)

    @pl.when(r == 0)
    def _():
        sacc[0:1] = s1
        sacc[1:2] = s2

    @pl.when(r > 0)
    def _():
        sacc[0:1] = sacc[0:1] + s1
        sacc[1:2] = sacc[1:2] + s2

    @pl.when(r == NR - 1)
    def _():
        st_ref[0] = sacc[0:2]


def _post23_call(agg, bcr, degpR):
    return pl.pallas_call(
        _post23_body,
        grid=(4, NR),
        in_specs=[
            pl.BlockSpec((1, 1, RB, 128), lambda co, r: (co, r, 0, 0)),
            pl.BlockSpec((1, 1, 128), lambda co, r: (co, 0, 0)),
            pl.BlockSpec((1, RB, 2), lambda co, r: (r, 0, 0)),
        ],
        out_specs=(
            pl.BlockSpec((1, 1, RB, 128), lambda co, r: (co, r, 0, 0)),
            pl.BlockSpec((1, 2, 128), lambda co, r: (co, 0, 0)),
        ),
        out_shape=(
            jax.ShapeDtypeStruct((4, NR, RB, 128), f32),
            jax.ShapeDtypeStruct((4, 2, 128), f32),
        ),
        scratch_shapes=[pltpu.VMEM((8, 128), f32)],
    )(agg, bcr, degpR)


def _final_body(y_ref, st_ref, g_ref, be_ref, x1_ref, wf1_ref, bf1_ref,
                wf2_ref, bf2_ref, o_ref):
    parts = []
    for cc in range(4):
        st = st_ref[cc]
        m = st[0:1] * (1.0 / N)
        v = st[1:2] * (1.0 / N) - m * m
        a = g_ref[cc] * lax.rsqrt(v + 1e-5)
        cs = be_ref[cc] - m * a
        yb = y_ref[cc, 0]
        hg_c = a * (jnp.sum(yb, axis=0, keepdims=True) * (1.0 / NPG)) + cs
        parts.append(hg_c)
    hg = jnp.concatenate(parts, axis=1)
    z = jnp.concatenate([hg, x1_ref[0]], axis=1)
    z = jnp.maximum(_doth(z, wf1_ref[...]) + bf1_ref[...], 0.0)
    z = _doth(z, wf2_ref[...]) + bf2_ref[...]
    mz = jnp.max(z, axis=1, keepdims=True)
    lse = jnp.log(jnp.sum(jnp.exp(z - mz), axis=1, keepdims=True)) + mz
    o_ref[0] = z - lse


def _final_call(y3g, st3, g3r, be3r, x1, wf1, bf1r, wf2, bf2r):
    return pl.pallas_call(
        _final_body,
        grid=(B,),
        in_specs=[
            pl.BlockSpec((4, 1, NPG, 128), lambda b: (0, b, 0, 0)),
            pl.BlockSpec((4, 2, 128), lambda b: (0, 0, 0)),
            pl.BlockSpec((4, 1, 128), lambda b: (0, 0, 0)),
            pl.BlockSpec((4, 1, 128), lambda b: (0, 0, 0)),
            pl.BlockSpec((1, 1, F), lambda b: (b, 0, 0)),
            pl.BlockSpec((D + F, D), lambda b: (0, 0)),
            pl.BlockSpec((1, D), lambda b: (0, 0)),
            pl.BlockSpec((D, C), lambda b: (0, 0)),
            pl.BlockSpec((1, C), lambda b: (0, 0)),
        ],
        out_specs=pl.BlockSpec((1, 1, C), lambda b: (b, 0, 0)),
        out_shape=jax.ShapeDtypeStruct((B, 1, C), f32),
    )(y3g, st3, g3r, be3r, x1, wf1, bf1r, wf2, bf2r)


@functools.cache
def _get_deg_kernel():
    return _make_deg_kernel()


@functools.cache
def _get_scatter_kernel(nch):
    return _make_scatter_kernel(nch)


def kernel(x, edge_index, batch, y, degree, closeness, Wp1, Wp2, Wsm, bsm,
           Wc1, bc1, g1, be1, Wc2, bc2, g2, be2, Wc3, bc3, g3, be3,
           Wf1, bf1, Wf2, bf2):
    x3 = x.reshape(B, NPG, F)
    clo3 = closeness.reshape(B, NPG, 1)
    degin3 = degree.reshape(B, NPG, 1)
    wp2r = Wp2.reshape(1, F)

    score = _score_call(x3, clo3, degin3, Wp1, wp2r)
    scp = jnp.concatenate(
        [score.reshape(B, NPG), jnp.full((B, 12), -3e38, f32)], axis=1)
    kn, idx4 = _top4_call(scp, x)

    src = edge_index[0]
    dst = edge_index[1]
    dst32 = dst.reshape(NCORE * NSUB, DBLK, EB)
    ones_eb = jnp.ones((EB,), f32)
    zeros_n = jnp.zeros((N,), f32)
    degp = _get_deg_kernel()(dst32, ones_eb, zeros_n)
    degp2 = jnp.moveaxis(degp, 0, 1)
    degpB = degp2.reshape(B, NPG, 2)
    degpR = degp2.reshape(NR, RB, 2)

    crow = Wsm[0:1] + bsm[None, :]
    wx = Wsm[1:1 + F]
    w0 = Wsm[1 + F:1 + 2 * F]
    w1 = Wsm[1 + 2 * F:1 + 3 * F]
    w2 = Wsm[1 + 3 * F:1 + 4 * F]
    h1p, x1 = _spline_call(x3, kn, idx4, degpB, crow, wx, w0, w1, w2)

    dst16 = dst.reshape(NSUB, 2, NBLK2, EB)
    srcg2 = (src[None, :].astype(i32)
             + (jnp.arange(2, dtype=i32) * N)[:, None]).reshape(
                 2, NSUB, 2, NBLK2, EB)
    srcg4 = (src[None, :].astype(i32)
             + (jnp.arange(4, dtype=i32) * N)[:, None]).reshape(
                 4, NSUB, 2, NBLK2, EB)

    agg1 = _get_scatter_kernel(2)(h1p.reshape(2 * N, 128), srcg2, dst16)
    agg1 = agg1.reshape(2, NR, RB, 128)

    wc1r = Wc1.reshape(2, 128, 4, 128).transpose(0, 2, 1, 3)
    bc1r = bc1.reshape(4, 1, 128)
    y1, st1 = _post1_call(agg1, degpR, wc1r, bc1r)

    wc2r = Wc2.reshape(4, 128, 4, 128).transpose(0, 2, 1, 3)
    h2p2 = _mm_call(y1, st1, g1.reshape(4, 1, 128), be1.reshape(4, 1, 128),
                    degpR, wc2r)
    agg2 = _get_scatter_kernel(4)(h2p2.reshape(4 * N, 128), srcg4, dst16)
    y2, st2 = _post23_call(agg2.reshape(4, NR, RB, 128),
                           bc2.reshape(4, 1, 128), degpR)

    wc3r = Wc3.reshape(4, 128, 4, 128).transpose(0, 2, 1, 3)
    h2p3 = _mm_call(y2, st2, g2.reshape(4, 1, 128), be2.reshape(4, 1, 128),
                    degpR, wc3r)
    agg3 = _get_scatter_kernel(4)(h2p3.reshape(4 * N, 128), srcg4, dst16)
    y3, st3 = _post23_call(agg3.reshape(4, NR, RB, 128),
                           bc3.reshape(4, 1, 128), degpR)

    out = _final_call(y3.reshape(4, B, NPG, 128), st3,
                      g3.reshape(4, 1, 128), be3.reshape(4, 1, 128),
                      x1, Wf1, bf1.reshape(1, D), Wf2, bf2.reshape(1, C))
    return out.reshape(B, C)
